# Initial kernel scaffold; baseline (speedup 1.0000x reference)
import jax
import jax.numpy as jnp
from jax.experimental import pallas as pl


def kernel(topics, structure_abstracts, contextual_table, structural_table):
    raise NotImplementedError("write your pallas kernel here")


# speedup vs baseline: 3.9968x; 3.9968x over previous
import functools

import jax
import jax.numpy as jnp
from jax import lax
from jax.experimental import pallas as pl
from jax.experimental.pallas import tpu as pltpu
from jax.experimental.pallas import tpu_sc as plsc

NUM_CONTEXTS = 1000000
CTX_DIM = 64
BATCH = 16384
N_FIELDS = 26
HIST = 200

NC = 2
NS = 16
L = 16
NW = NC * NS

B1 = BATCH * N_FIELDS
B2 = BATCH * HIST
PER_W1 = B1 // NW
PER_W2 = B2 // NW

CC = 512
N_CHUNKS1 = PER_W1 // CC
GB = 128
N_BURSTS = CC // GB

SCCH = 2048
N_CHUNKS2 = PER_W2 // SCCH


def _tanh16(x):
    e = jnp.exp(jnp.abs(x) * -2.0)
    return (1.0 - e) / (1.0 + e) * jnp.sign(x)


def _worker_id():
    return lax.axis_index("s") * NC + lax.axis_index("c")


def _gather_bursts(table, idx_v, rows_v, sem):
    cps = [pltpu.async_copy(table.at[idx_v.at[j]],
                            rows_v.at[pl.ds(j * GB, GB)], sem)
           for j in range(N_BURSTS)]
    for cp in cps:
        cp.wait()


def _sc_body(topics2d, sa_flat, table, stable_pad, out1, out2,
             idx_v, rows_v, sidx_v, obuf, tt2, tt, sem):
    wid = _worker_id()
    ii = lax.iota(jnp.int32, L)

    pltpu.sync_copy(stable_pad, tt2)
    tt[...] = _tanh16(tt2[...])

    row0 = wid * (PER_W1 // GB)

    @pl.loop(0, N_CHUNKS1)
    def _ctx_chunk(g):
        r0 = row0 + g * N_BURSTS
        pltpu.sync_copy(topics2d.at[pl.ds(r0, N_BURSTS)], idx_v)
        _gather_bursts(table, idx_v, rows_v, sem)

        @pl.loop(0, CC)
        def _row(i):
            for s in range(CTX_DIM // L):
                sl = pl.ds(s * L, L)
                rows_v[i, sl] = _tanh16(rows_v[i, sl])

        pltpu.sync_copy(rows_v, out1.at[pl.ds(wid * PER_W1 + g * CC, CC)])

    sbase = wid * PER_W2

    @pl.loop(0, N_CHUNKS2)
    def _str_chunk(g):
        pltpu.sync_copy(sa_flat.at[pl.ds(sbase + g * SCCH, SCCH)], sidx_v)

        @pl.loop(0, SCCH // L)
        def _grp(j):
            iv = sidx_v[pl.ds(j * L, L)]
            g0 = iv * 2
            v0 = plsc.load_gather(tt, [g0])
            v1 = plsc.load_gather(tt, [g0 + 1])
            pos = j * (2 * L) + ii * 2
            plsc.store_scatter(obuf, [pos], v0)
            plsc.store_scatter(obuf, [pos + 1], v1)

        pltpu.sync_copy(obuf, out2.at[pl.ds(2 * (sbase + g * SCCH), 2 * SCCH)])


@jax.jit
def _run(topics2d, sa_flat, table, stable_pad):
    mesh = plsc.VectorSubcoreMesh(core_axis_name="c", subcore_axis_name="s",
                                  num_cores=NC, num_subcores=NS)
    f = pl.kernel(
        _sc_body,
        out_type=[
            jax.ShapeDtypeStruct((B1, CTX_DIM), jnp.float32),
            jax.ShapeDtypeStruct((2 * B2,), jnp.float32),
        ],
        mesh=mesh,
        compiler_params=pltpu.CompilerParams(needs_layout_passes=False,
                                             use_tc_tiling_on_sc=False),
        scratch_types=[
            pltpu.VMEM((N_BURSTS, GB), jnp.int32),
            pltpu.VMEM((CC, CTX_DIM), jnp.float32),
            pltpu.VMEM((SCCH,), jnp.int32),
            pltpu.VMEM((2 * SCCH,), jnp.float32),
            pltpu.VMEM((L,), jnp.float32),
            pltpu.VMEM((L,), jnp.float32),
            pltpu.SemaphoreType.DMA,
        ],
    )
    return f(topics2d, sa_flat, table, stable_pad)


def kernel(topics, structure_abstracts, contextual_table, structural_table):
    topics2d = topics.reshape(B1 // GB, GB)
    sa_flat = structure_abstracts.reshape(B2)
    stable_pad = jnp.pad(structural_table.reshape(6), (0, L - 6))
    out1, out2 = _run(topics2d, sa_flat, contextual_table, stable_pad)
    return (out1.reshape(BATCH, 1, N_FIELDS * CTX_DIM),
            out2.reshape(BATCH, HIST, 2))


# speedup vs baseline: 13.5413x; 3.3880x over previous
import functools

import jax
import jax.numpy as jnp
from jax import lax
from jax.experimental import pallas as pl
from jax.experimental.pallas import tpu as pltpu
from jax.experimental.pallas import tpu_sc as plsc

NUM_CONTEXTS = 1000000
CTX_DIM = 64
BATCH = 16384
N_FIELDS = 26
HIST = 200

NC = 2
NS = 16
L = 16
NW = NC * NS

B1 = BATCH * N_FIELDS
PER_W1 = B1 // NW

CC = 512
N_CHUNKS1 = PER_W1 // CC
GB = 128
N_BURSTS = CC // GB

SQ = BATCH // 4
UNITS_PER_W = HIST * 4 // NW


def _tanh16(x):
    e = jnp.exp(jnp.abs(x) * -2.0)
    t = (1.0 - e) / (1.0 + e)
    tb = plsc.bitcast(t, jnp.int32)
    sb = plsc.bitcast(x, jnp.int32) & jnp.int32(-2147483648)
    return plsc.bitcast(tb | sb, jnp.float32)


def _worker_id():
    return lax.axis_index("s") * NC + lax.axis_index("c")


def _gather_bursts(table, idx_v, rows_v, sem):
    cps = [pltpu.async_copy(table.at[idx_v.at[j]],
                            rows_v.at[pl.ds(j * GB, GB)], sem)
           for j in range(N_BURSTS)]
    for cp in cps:
        cp.wait()


def _sc_body(topics2d, sa_hb, table, stable_pad, out1, out2,
             idx_v, rows_v, sidx_v, obuf, tt2, tt, sem):
    wid = _worker_id()

    pltpu.sync_copy(stable_pad, tt2)
    tt[...] = _tanh16(tt2[...])

    row0 = wid * (PER_W1 // GB)

    @pl.loop(0, N_CHUNKS1)
    def _ctx_chunk(g):
        r0 = row0 + g * N_BURSTS
        pltpu.sync_copy(topics2d.at[pl.ds(r0, N_BURSTS)], idx_v)
        _gather_bursts(table, idx_v, rows_v, sem)

        @pl.loop(0, CC, unroll=8)
        def _row(i):
            for s in range(CTX_DIM // L):
                sl = pl.ds(s * L, L)
                rows_v[i, sl] = _tanh16(rows_v[i, sl])

        pltpu.sync_copy(rows_v, out1.at[pl.ds(wid * PER_W1 + g * CC, CC)])

    @pl.loop(0, UNITS_PER_W)
    def _str_unit(u):
        g = wid * UNITS_PER_W + u
        h = g >> 2
        q = g & 3
        pltpu.sync_copy(sa_hb.at[pl.ds(h * BATCH + q * SQ, SQ)], sidx_v)

        @pl.loop(0, SQ // L, unroll=8)
        def _grp(j):
            sl = pl.ds(j * L, L)
            g0 = sidx_v[sl] * 2
            obuf[0, sl] = plsc.load_gather(tt, [g0])
            obuf[1, sl] = plsc.load_gather(tt, [g0 + 1])

        ob = (2 * h) * BATCH + q * SQ
        pltpu.sync_copy(obuf.at[0], out2.at[pl.ds(ob, SQ)])
        pltpu.sync_copy(obuf.at[1], out2.at[pl.ds(ob + BATCH, SQ)])


@jax.jit
def _run(topics2d, sa_hb, table, stable_pad):
    mesh = plsc.VectorSubcoreMesh(core_axis_name="c", subcore_axis_name="s",
                                  num_cores=NC, num_subcores=NS)
    f = pl.kernel(
        _sc_body,
        out_type=[
            jax.ShapeDtypeStruct((B1, CTX_DIM), jnp.float32),
            jax.ShapeDtypeStruct((HIST * 2 * BATCH,), jnp.float32),
        ],
        mesh=mesh,
        compiler_params=pltpu.CompilerParams(needs_layout_passes=False,
                                             use_tc_tiling_on_sc=False),
        scratch_types=[
            pltpu.VMEM((N_BURSTS, GB), jnp.int32),
            pltpu.VMEM((CC, CTX_DIM), jnp.float32),
            pltpu.VMEM((SQ,), jnp.int32),
            pltpu.VMEM((2, SQ), jnp.float32),
            pltpu.VMEM((L,), jnp.float32),
            pltpu.VMEM((L,), jnp.float32),
            pltpu.SemaphoreType.DMA,
        ],
    )
    return f(topics2d, sa_hb, table, stable_pad)


def kernel(topics, structure_abstracts, contextual_table, structural_table):
    topics2d = topics.reshape(B1 // GB, GB)
    sa_hb = structure_abstracts.T.reshape(HIST * BATCH)
    stable_pad = jnp.pad(structural_table.reshape(6), (0, L - 6))
    out1, out2 = _run(topics2d, sa_hb, contextual_table, stable_pad)
    return (out1.reshape(BATCH, 1, N_FIELDS * CTX_DIM),
            out2.reshape(HIST, 2, BATCH).transpose(2, 0, 1))


# speedup vs baseline: 14.8305x; 1.0952x over previous
import functools

import jax
import jax.numpy as jnp
from jax import lax
from jax.experimental import pallas as pl
from jax.experimental.pallas import tpu as pltpu
from jax.experimental.pallas import tpu_sc as plsc

NUM_CONTEXTS = 1000000
CTX_DIM = 64
BATCH = 16384
N_FIELDS = 26
HIST = 200

NC = 2
NS = 16
L = 16
NW = NC * NS

B1 = BATCH * N_FIELDS
PER_W1 = B1 // NW

CC = 512
N_CHUNKS1 = PER_W1 // CC
GB = 128
N_BURSTS = CC // GB

SQ = BATCH // 4
UNITS_PER_W = HIST * 4 // NW


def _tanh16(x):
    e = jnp.exp(jnp.abs(x) * -2.0)
    t = (1.0 - e) / (1.0 + e)
    tb = plsc.bitcast(t, jnp.int32)
    sb = plsc.bitcast(x, jnp.int32) & jnp.int32(-2147483648)
    return plsc.bitcast(tb | sb, jnp.float32)


def _worker_id():
    return lax.axis_index("s") * NC + lax.axis_index("c")


def _fire_bursts(table, idx_v, rows_v, b, sem):
    for j in range(N_BURSTS):
        pltpu.async_copy(table.at[idx_v.at[b, j]],
                         rows_v.at[b, pl.ds(j * GB, GB)], sem)


def _wait_bursts(table, idx_v, rows_v, b, sem):
    for j in range(N_BURSTS):
        pltpu.make_async_copy(table.at[idx_v.at[b, j]],
                              rows_v.at[b, pl.ds(j * GB, GB)], sem).wait()


def _sc_body(topics2d, sa4, table, stable_pad, out1, out2,
             idx_v, rows_v, sidx_v, obuf, tt2, tt, sem):
    wid = _worker_id()

    pltpu.sync_copy(stable_pad, tt2)
    tt[...] = _tanh16(tt2[...])

    row0 = wid * (PER_W1 // GB)

    pltpu.sync_copy(topics2d.at[pl.ds(row0, N_BURSTS)], idx_v.at[0])
    _fire_bursts(table, idx_v, rows_v, 0, sem)

    @pl.loop(0, N_CHUNKS1)
    def _ctx_chunk(g):
        b = lax.rem(g, 2)
        nb = 1 - b

        @pl.when(g + 1 < N_CHUNKS1)
        def _prefetch():
            r0 = row0 + (g + 1) * N_BURSTS
            pltpu.sync_copy(topics2d.at[pl.ds(r0, N_BURSTS)], idx_v.at[nb])
            _fire_bursts(table, idx_v, rows_v, nb, sem)

        _wait_bursts(table, idx_v, rows_v, b, sem)

        @pl.loop(0, CC, unroll=8)
        def _row(i):
            for s in range(CTX_DIM // L):
                sl = pl.ds(s * L, L)
                rows_v[b, i, sl] = _tanh16(rows_v[b, i, sl])

        pltpu.sync_copy(rows_v.at[b],
                        out1.at[pl.ds(wid * PER_W1 + g * CC, CC)])

    @pl.loop(0, UNITS_PER_W)
    def _str_unit(u):
        g = wid * UNITS_PER_W + u
        h = g >> 2
        q = g & 3
        tr = h >> 3
        r = h & 7
        pltpu.sync_copy(sa4.at[tr, pl.ds(q * 32, 32), r], sidx_v)

        @pl.loop(0, 32)
        def _grp(i):
            for l in range(GB // L):
                g0 = sidx_v[i, pl.ds(l * L, L)] * 2
                sl = pl.ds(i * GB + l * L, L)
                obuf[0, sl] = plsc.load_gather(tt, [g0])
                obuf[1, sl] = plsc.load_gather(tt, [g0 + 1])

        ob = (2 * h) * BATCH + q * SQ
        pltpu.sync_copy(obuf.at[0], out2.at[pl.ds(ob, SQ)])
        pltpu.sync_copy(obuf.at[1], out2.at[pl.ds(ob + BATCH, SQ)])


@jax.jit
def _run(topics2d, sa4, table, stable_pad):
    mesh = plsc.VectorSubcoreMesh(core_axis_name="c", subcore_axis_name="s",
                                  num_cores=NC, num_subcores=NS)
    f = pl.kernel(
        _sc_body,
        out_type=[
            jax.ShapeDtypeStruct((B1, CTX_DIM), jnp.float32),
            jax.ShapeDtypeStruct((HIST * 2 * BATCH,), jnp.float32),
        ],
        mesh=mesh,
        compiler_params=pltpu.CompilerParams(needs_layout_passes=False,
                                             use_tc_tiling_on_sc=False),
        scratch_types=[
            pltpu.VMEM((2, N_BURSTS, GB), jnp.int32),
            pltpu.VMEM((2, CC, CTX_DIM), jnp.float32),
            pltpu.VMEM((32, GB), jnp.int32),
            pltpu.VMEM((2, SQ), jnp.float32),
            pltpu.VMEM((L,), jnp.float32),
            pltpu.VMEM((L,), jnp.float32),
            pltpu.SemaphoreType.DMA,
        ],
    )
    return f(topics2d, sa4, table, stable_pad)


def kernel(topics, structure_abstracts, contextual_table, structural_table):
    topics2d = topics.reshape(B1 // GB, GB)
    sa4 = structure_abstracts.reshape(128, 128, 25, 8).transpose(2, 0, 3, 1)
    stable_pad = jnp.pad(structural_table.reshape(6), (0, L - 6))
    out1, out2 = _run(topics2d, sa4, contextual_table, stable_pad)
    return (out1.reshape(BATCH, 1, N_FIELDS * CTX_DIM),
            out2.reshape(HIST, 2, BATCH).transpose(2, 0, 1))
